# baseline (speedup 1.0000x reference)
import functools

import jax
import jax.numpy as jnp
from jax import lax
from jax.experimental import pallas as pl
from jax.experimental.pallas import tpu as pltpu
from jax.experimental.pallas import tpu_sc as plsc

N = 10000
NP = 10240
D = 128
E = 320000
EP = 327680
NC = 2
NS = 16
NW = NC * NS
CH = 128
EPT = EP // NW
NCHA = EPT // CH
CHC = 64
NT = EPT // CHC
SI = 10
NST = NT // SI
NB = 4
NBE = 8
RPT = NP // NS

_mesh = plsc.VectorSubcoreMesh(
    core_axis_name="c", subcore_axis_name="s", num_cores=NC, num_subcores=NS)


@functools.partial(
    pl.kernel,
    out_type=jax.ShapeDtypeStruct((NC, NP), jnp.float32),
    mesh=_mesh,
    scratch_types=[
        pltpu.VMEM((NCHA, CH), jnp.int32),
        pltpu.VMEM((CH,), jnp.float32),
        pltpu.VMEM((RPT,), jnp.float32),
        pltpu.VMEM_SHARED((NP,), jnp.float32),
        pltpu.SemaphoreType.DMA,
    ],
)
def _sc_count(dst_h, cnt_h, idx_v, ones_v, zv, cnt_sh, sem_s):
    cid = lax.axis_index("c")
    sid = lax.axis_index("s")
    wid = cid * NS + sid
    for i in range(CH // 16):
        ones_v[pl.ds(i * 16, 16)] = jnp.ones((16,), jnp.float32)
    for i in range(RPT // 16):
        zv[pl.ds(i * 16, 16)] = jnp.zeros((16,), jnp.float32)
    pltpu.sync_copy(zv, cnt_sh.at[pl.ds(sid * RPT, RPT)])
    plsc.subcore_barrier()
    pltpu.sync_copy(dst_h.at[wid], idx_v)

    def chunk(c, carry):
        @pl.when(c >= 8)
        def _drain():
            pltpu.make_async_copy(ones_v, cnt_sh.at[idx_v.at[0]],
                                  sem_s).wait()

        pltpu.async_copy(ones_v, cnt_sh.at[idx_v.at[c]], sem_s, add=True)
        return carry

    lax.fori_loop(0, NCHA, chunk, 0)
    for _ in range(8):
        pltpu.make_async_copy(ones_v, cnt_sh.at[idx_v.at[0]], sem_s).wait()
    plsc.subcore_barrier()
    pltpu.sync_copy(cnt_sh.at[pl.ds(sid * RPT, RPT)],
                    cnt_h.at[cid, pl.ds(sid * RPT, RPT)])


@functools.partial(
    pl.kernel,
    out_type=jax.ShapeDtypeStruct((NC, NP, D), jnp.float32),
    mesh=_mesh,
    scratch_types=[
        pltpu.VMEM((2, SI, CHC), jnp.int32),
        pltpu.VMEM((2, SI, CHC), jnp.int32),
        pltpu.VMEM((NB, CHC, D), jnp.float32),
        pltpu.VMEM_SHARED((NP, D), jnp.float32),
        pltpu.SemaphoreType.DMA,
        pltpu.SemaphoreType.DMA,
        pltpu.SemaphoreType.DMA,
    ],
)
def _sc_agg(src_h, dst_h, g_h, acc_h, isrc, idst, rows, acc_sh,
            sem_g, sem_s, sem_i):
    cid = lax.axis_index("c")
    sid = lax.axis_index("s")
    wid = cid * NS + sid

    def zbody(i, carry):
        for j in range(D // 16):
            rows[0, i, pl.ds(j * 16, 16)] = jnp.zeros((16,), jnp.float32)
        return carry

    lax.fori_loop(0, CHC, zbody, 0)
    for k in range(RPT // CHC):
        pltpu.sync_copy(rows.at[0], acc_sh.at[pl.ds(sid * RPT + k * CHC, CHC)])
    plsc.subcore_barrier()

    pltpu.sync_copy(src_h.at[wid, 0], isrc.at[0])
    pltpu.sync_copy(dst_h.at[wid, 0], idst.at[0])
    pltpu.async_copy(g_h.at[isrc.at[0, 0]], rows.at[0], sem_g)
    pltpu.async_copy(g_h.at[isrc.at[0, 1]], rows.at[1], sem_g)

    def step(c, carry):
        st = c // SI
        cc = c % SI
        sb = st % 2

        @pl.when(jnp.logical_and(cc == 0, st + 1 < NST))
        def _pref():
            pltpu.async_copy(src_h.at[wid, st + 1],
                             isrc.at[(st + 1) % 2], sem_i)
            pltpu.async_copy(dst_h.at[wid, st + 1],
                             idst.at[(st + 1) % 2], sem_i)

        @pl.when(c >= 2)
        def _drain():
            pltpu.make_async_copy(rows.at[0], acc_sh.at[idst.at[0, 0]],
                                  sem_s).wait()

        pltpu.make_async_copy(g_h.at[pl.ds(0, CHC)], rows.at[c % NB],
                              sem_g).wait()

        @pl.when(jnp.logical_and(cc == SI - 2, st + 1 < NST))
        def _wslab():
            pltpu.make_async_copy(src_h.at[0, 0], isrc.at[0], sem_i).wait()
            pltpu.make_async_copy(dst_h.at[0, 0], idst.at[0], sem_i).wait()

        @pl.when(c + 2 < NT)
        def _gnext():
            c2 = c + 2
            pltpu.async_copy(
                g_h.at[isrc.at[(c2 // SI) % 2, c2 % SI]],
                rows.at[c2 % NB], sem_g)

        pltpu.async_copy(rows.at[c % NB], acc_sh.at[idst.at[sb, cc]],
                         sem_s, add=True)
        return carry

    lax.fori_loop(0, NT, step, 0)
    for _ in range(2):
        pltpu.make_async_copy(rows.at[0], acc_sh.at[idst.at[0, 0]],
                              sem_s).wait()
    plsc.subcore_barrier()
    pltpu.sync_copy(acc_sh.at[pl.ds(sid * RPT, RPT)],
                    acc_h.at[cid, pl.ds(sid * RPT, RPT)])


@functools.partial(
    pl.kernel,
    out_type=jax.ShapeDtypeStruct((NC, NP), jnp.float32),
    mesh=_mesh,
    scratch_types=[
        pltpu.VMEM((NCHA, CH), jnp.int32),
        pltpu.VMEM((NCHA, CH), jnp.int32),
        pltpu.VMEM((NBE, CH), jnp.float32),
        pltpu.VMEM((RPT,), jnp.float32),
        pltpu.VMEM_SHARED((NP,), jnp.float32),
        pltpu.SemaphoreType.DMA,
        pltpu.SemaphoreType.DMA,
    ],
)
def _sc_agg1(src_h, dst_h, t_h, acc_h, isrc, idst, tv, zv, acc_sh,
             sem_g, sem_s):
    cid = lax.axis_index("c")
    sid = lax.axis_index("s")
    wid = cid * NS + sid
    for i in range(RPT // 16):
        zv[pl.ds(i * 16, 16)] = jnp.zeros((16,), jnp.float32)
    pltpu.sync_copy(zv, acc_sh.at[pl.ds(sid * RPT, RPT)])
    plsc.subcore_barrier()
    pltpu.sync_copy(src_h.at[wid], isrc)
    pltpu.sync_copy(dst_h.at[wid], idst)
    pltpu.async_copy(t_h.at[isrc.at[0]], tv.at[0], sem_g)
    pltpu.async_copy(t_h.at[isrc.at[1]], tv.at[1], sem_g)

    def step(c, carry):
        @pl.when(c >= NBE - 2)
        def _drain():
            pltpu.make_async_copy(tv.at[0], acc_sh.at[idst.at[0]],
                                  sem_s).wait()

        pltpu.make_async_copy(t_h.at[pl.ds(0, CH)], tv.at[c % NBE],
                              sem_g).wait()

        @pl.when(c + 2 < NCHA)
        def _gnext():
            pltpu.async_copy(t_h.at[isrc.at[c + 2]], tv.at[(c + 2) % NBE],
                             sem_g)

        pltpu.async_copy(tv.at[c % NBE], acc_sh.at[idst.at[c]],
                         sem_s, add=True)
        return carry

    lax.fori_loop(0, NCHA, step, 0)
    for _ in range(NBE - 2):
        pltpu.make_async_copy(tv.at[0], acc_sh.at[idst.at[0]], sem_s).wait()
    plsc.subcore_barrier()
    pltpu.sync_copy(acc_sh.at[pl.ds(sid * RPT, RPT)],
                    acc_h.at[cid, pl.ds(sid * RPT, RPT)])


BM = 640
GRID = NP // BM


def _tc_b1_body(x_ref, w1_ref, h_ref):
    h_ref[...] = jnp.dot(x_ref[...], w1_ref[...],
                         preferred_element_type=jnp.float32)


def _tc_b1(xp, w1):
    return pl.pallas_call(
        _tc_b1_body,
        grid=(GRID,),
        in_specs=[
            pl.BlockSpec((BM, D), lambda i: (i, 0)),
            pl.BlockSpec((D, D), lambda i: (0, 0)),
        ],
        out_specs=pl.BlockSpec((BM, D), lambda i: (i, 0)),
        out_shape=jax.ShapeDtypeStruct((NP, D), jnp.float32),
    )(xp, w1)


def _tc_b2_body(h_ref, c0_ref, c1_ref, g_ref, dinv_ref):
    cnt = c0_ref[...] + c1_ref[...]
    dinv = lax.rsqrt(cnt + 1.0)
    g_ref[...] = h_ref[...] * dinv
    dinv_ref[...] = dinv


def _tc_b2(h, c0, c1):
    return pl.pallas_call(
        _tc_b2_body,
        grid=(GRID,),
        in_specs=[
            pl.BlockSpec((BM, D), lambda i: (i, 0)),
            pl.BlockSpec((BM, 1), lambda i: (i, 0)),
            pl.BlockSpec((BM, 1), lambda i: (i, 0)),
        ],
        out_specs=[
            pl.BlockSpec((BM, D), lambda i: (i, 0)),
            pl.BlockSpec((BM, 1), lambda i: (i, 0)),
        ],
        out_shape=[
            jax.ShapeDtypeStruct((NP, D), jnp.float32),
            jax.ShapeDtypeStruct((NP, 1), jnp.float32),
        ],
    )(h, c0, c1)


def _tc_d_body(a0_ref, a1_ref, g_ref, dinv_ref, b1_ref, w2_ref, t_ref):
    acc = a0_ref[...] + a1_ref[...] + g_ref[...]
    out1 = acc * dinv_ref[...] + b1_ref[...]
    h1 = jnp.maximum(out1, 0.0)
    s = jnp.dot(h1, w2_ref[...], preferred_element_type=jnp.float32)
    t_ref[...] = s * dinv_ref[...]


def _tc_d(a0, a1, g, dinv, b1r, w2):
    return pl.pallas_call(
        _tc_d_body,
        grid=(GRID,),
        in_specs=[
            pl.BlockSpec((BM, D), lambda i: (i, 0)),
            pl.BlockSpec((BM, D), lambda i: (i, 0)),
            pl.BlockSpec((BM, D), lambda i: (i, 0)),
            pl.BlockSpec((BM, 1), lambda i: (i, 0)),
            pl.BlockSpec((1, D), lambda i: (0, 0)),
            pl.BlockSpec((D, 1), lambda i: (0, 0)),
        ],
        out_specs=pl.BlockSpec((BM, 1), lambda i: (i, 0)),
        out_shape=jax.ShapeDtypeStruct((NP, 1), jnp.float32),
    )(a0, a1, g, dinv, b1r, w2)


def _tc_f_body(q0_ref, q1_ref, t_ref, dinv_ref, b2_ref, o_ref):
    o_ref[...] = (q0_ref[...] + q1_ref[...] + t_ref[...]) * dinv_ref[...] \
        + b2_ref[...]


def _tc_f(q0, q1, t, dinv, b2r):
    return pl.pallas_call(
        _tc_f_body,
        grid=(GRID,),
        in_specs=[
            pl.BlockSpec((BM, 1), lambda i: (i, 0)),
            pl.BlockSpec((BM, 1), lambda i: (i, 0)),
            pl.BlockSpec((BM, 1), lambda i: (i, 0)),
            pl.BlockSpec((BM, 1), lambda i: (i, 0)),
            pl.BlockSpec((1, 1), lambda i: (0, 0)),
        ],
        out_specs=pl.BlockSpec((BM, 1), lambda i: (i, 0)),
        out_shape=jax.ShapeDtypeStruct((NP, 1), jnp.float32),
    )(q0, q1, t, dinv, b2r)


def kernel(x, edge_index, edge_attr, W1, b1, W2, b2):
    ei = edge_index.astype(jnp.int32)
    pad = N + jnp.arange(EP - E, dtype=jnp.int32) % (NP - N)
    srcp = jnp.concatenate([ei[0], pad])
    dstp = jnp.concatenate([ei[1], pad])
    src_a = srcp.reshape(NW, NCHA, CH)
    dst_a = dstp.reshape(NW, NCHA, CH)
    src_c = srcp.reshape(NW, NST, SI, CHC)
    dst_c = dstp.reshape(NW, NST, SI, CHC)
    xp = jnp.zeros((NP, D), jnp.float32).at[:N].set(x)

    h = _tc_b1(xp, W1)
    cnt = _sc_count(dst_a)
    c0 = cnt[0].reshape(NP, 1)
    c1 = cnt[1].reshape(NP, 1)
    g, dinv = _tc_b2(h, c0, c1)
    acc = _sc_agg(src_c, dst_c, g)
    t = _tc_d(acc[0], acc[1], g, dinv, b1.reshape(1, D), W2)
    q = _sc_agg1(src_a, dst_a, t.reshape(NP))
    out = _tc_f(q[0].reshape(NP, 1), q[1].reshape(NP, 1), t, dinv,
                b2.reshape(1, 1))
    return out[:N]


# speedup vs baseline: 1.0400x; 1.0400x over previous
import functools

import jax
import jax.numpy as jnp
from jax import lax
from jax.experimental import pallas as pl
from jax.experimental.pallas import tpu as pltpu
from jax.experimental.pallas import tpu_sc as plsc

N = 10000
NP = 10240
D = 128
E = 320000
EP = 327680
NC = 2
NS = 16
NW = NC * NS
CH = 128
EPT = EP // NW
NCHA = EPT // CH
CHC = 64
NT = EPT // CHC
SI = 10
NST = NT // SI
NB = 4
NBE = 8
RPT = NP // NS

_mesh = plsc.VectorSubcoreMesh(
    core_axis_name="c", subcore_axis_name="s", num_cores=NC, num_subcores=NS)


@functools.partial(
    pl.kernel,
    out_type=jax.ShapeDtypeStruct((NC, NP), jnp.float32),
    mesh=_mesh,
    scratch_types=[
        pltpu.VMEM((NCHA, CH), jnp.int32),
        pltpu.VMEM((CH,), jnp.float32),
        pltpu.VMEM((RPT,), jnp.float32),
        pltpu.VMEM_SHARED((NP,), jnp.float32),
        pltpu.SemaphoreType.DMA,
    ],
)
def _sc_count(dst_h, cnt_h, idx_v, ones_v, zv, cnt_sh, sem_s):
    cid = lax.axis_index("c")
    sid = lax.axis_index("s")
    wid = cid * NS + sid
    for i in range(CH // 16):
        ones_v[pl.ds(i * 16, 16)] = jnp.ones((16,), jnp.float32)
    for i in range(RPT // 16):
        zv[pl.ds(i * 16, 16)] = jnp.zeros((16,), jnp.float32)
    pltpu.sync_copy(zv, cnt_sh.at[pl.ds(sid * RPT, RPT)])
    plsc.subcore_barrier()
    pltpu.sync_copy(dst_h.at[wid], idx_v)

    def chunk(c, carry):
        @pl.when(c >= 8)
        def _drain():
            pltpu.make_async_copy(ones_v, cnt_sh.at[idx_v.at[0]],
                                  sem_s).wait()

        pltpu.async_copy(ones_v, cnt_sh.at[idx_v.at[c]], sem_s, add=True)
        return carry

    lax.fori_loop(0, NCHA, chunk, 0)
    for _ in range(8):
        pltpu.make_async_copy(ones_v, cnt_sh.at[idx_v.at[0]], sem_s).wait()
    plsc.subcore_barrier()
    pltpu.sync_copy(cnt_sh.at[pl.ds(sid * RPT, RPT)],
                    cnt_h.at[cid, pl.ds(sid * RPT, RPT)])


@functools.partial(
    pl.kernel,
    out_type=jax.ShapeDtypeStruct((NC, NP, D), jnp.float32),
    mesh=_mesh,
    scratch_types=[
        pltpu.VMEM((2, SI, CHC), jnp.int32),
        pltpu.VMEM((2, SI, CHC), jnp.int32),
        pltpu.VMEM((NB, CHC, D), jnp.float32),
        pltpu.VMEM_SHARED((NP, D), jnp.float32),
        pltpu.SemaphoreType.DMA,
        pltpu.SemaphoreType.DMA,
        pltpu.SemaphoreType.DMA,
    ],
)
def _sc_agg(src_h, dst_h, g_h, acc_h, isrc, idst, rows, acc_sh,
            sem_g, sem_s, sem_i):
    cid = lax.axis_index("c")
    sid = lax.axis_index("s")
    wid = cid * NS + sid

    def zbody(i, carry):
        for j in range(D // 16):
            rows[0, i, pl.ds(j * 16, 16)] = jnp.zeros((16,), jnp.float32)
        return carry

    lax.fori_loop(0, CHC, zbody, 0)
    for k in range(RPT // CHC):
        pltpu.sync_copy(rows.at[0], acc_sh.at[pl.ds(sid * RPT + k * CHC, CHC)])
    plsc.subcore_barrier()

    pltpu.sync_copy(src_h.at[wid, 0], isrc.at[0])
    pltpu.sync_copy(dst_h.at[wid, 0], idst.at[0])
    pltpu.async_copy(g_h.at[isrc.at[0, 0]], rows.at[0], sem_g)
    pltpu.async_copy(g_h.at[isrc.at[0, 1]], rows.at[1], sem_g)

    def step(c, carry):
        st = c // SI
        cc = c % SI
        sb = st % 2

        @pl.when(jnp.logical_and(cc == 0, st + 1 < NST))
        def _pref():
            pltpu.async_copy(src_h.at[wid, st + 1],
                             isrc.at[(st + 1) % 2], sem_i)
            pltpu.async_copy(dst_h.at[wid, st + 1],
                             idst.at[(st + 1) % 2], sem_i)

        @pl.when(c >= 2)
        def _drain():
            pltpu.make_async_copy(rows.at[0], acc_sh.at[idst.at[0, 0]],
                                  sem_s).wait()

        pltpu.make_async_copy(g_h.at[pl.ds(0, CHC)], rows.at[c % NB],
                              sem_g).wait()

        @pl.when(jnp.logical_and(cc == SI - 2, st + 1 < NST))
        def _wslab():
            pltpu.make_async_copy(src_h.at[0, 0], isrc.at[0], sem_i).wait()
            pltpu.make_async_copy(dst_h.at[0, 0], idst.at[0], sem_i).wait()

        @pl.when(c + 2 < NT)
        def _gnext():
            c2 = c + 2
            pltpu.async_copy(
                g_h.at[isrc.at[(c2 // SI) % 2, c2 % SI]],
                rows.at[c2 % NB], sem_g)

        pltpu.async_copy(rows.at[c % NB], acc_sh.at[idst.at[sb, cc]],
                         sem_s, add=True)
        return carry

    lax.fori_loop(0, NT, step, 0, unroll=2)
    for _ in range(2):
        pltpu.make_async_copy(rows.at[0], acc_sh.at[idst.at[0, 0]],
                              sem_s).wait()
    plsc.subcore_barrier()
    pltpu.sync_copy(acc_sh.at[pl.ds(sid * RPT, RPT)],
                    acc_h.at[cid, pl.ds(sid * RPT, RPT)])


@functools.partial(
    pl.kernel,
    out_type=jax.ShapeDtypeStruct((NC, NP), jnp.float32),
    mesh=_mesh,
    scratch_types=[
        pltpu.VMEM((NCHA, CH), jnp.int32),
        pltpu.VMEM((NCHA, CH), jnp.int32),
        pltpu.VMEM((NBE, CH), jnp.float32),
        pltpu.VMEM((RPT,), jnp.float32),
        pltpu.VMEM_SHARED((NP,), jnp.float32),
        pltpu.SemaphoreType.DMA,
        pltpu.SemaphoreType.DMA,
    ],
)
def _sc_agg1(src_h, dst_h, t_h, acc_h, isrc, idst, tv, zv, acc_sh,
             sem_g, sem_s):
    cid = lax.axis_index("c")
    sid = lax.axis_index("s")
    wid = cid * NS + sid
    for i in range(RPT // 16):
        zv[pl.ds(i * 16, 16)] = jnp.zeros((16,), jnp.float32)
    pltpu.sync_copy(zv, acc_sh.at[pl.ds(sid * RPT, RPT)])
    plsc.subcore_barrier()
    pltpu.sync_copy(src_h.at[wid], isrc)
    pltpu.sync_copy(dst_h.at[wid], idst)
    pltpu.async_copy(t_h.at[isrc.at[0]], tv.at[0], sem_g)
    pltpu.async_copy(t_h.at[isrc.at[1]], tv.at[1], sem_g)

    def step(c, carry):
        @pl.when(c >= NBE - 2)
        def _drain():
            pltpu.make_async_copy(tv.at[0], acc_sh.at[idst.at[0]],
                                  sem_s).wait()

        pltpu.make_async_copy(t_h.at[pl.ds(0, CH)], tv.at[c % NBE],
                              sem_g).wait()

        @pl.when(c + 2 < NCHA)
        def _gnext():
            pltpu.async_copy(t_h.at[isrc.at[c + 2]], tv.at[(c + 2) % NBE],
                             sem_g)

        pltpu.async_copy(tv.at[c % NBE], acc_sh.at[idst.at[c]],
                         sem_s, add=True)
        return carry

    lax.fori_loop(0, NCHA, step, 0, unroll=4)
    for _ in range(NBE - 2):
        pltpu.make_async_copy(tv.at[0], acc_sh.at[idst.at[0]], sem_s).wait()
    plsc.subcore_barrier()
    pltpu.sync_copy(acc_sh.at[pl.ds(sid * RPT, RPT)],
                    acc_h.at[cid, pl.ds(sid * RPT, RPT)])


BM = 640
GRID = NP // BM


def _tc_b_body(x_ref, w1_ref, c0_ref, c1_ref, g_ref, dinv_ref):
    cnt = c0_ref[...] + c1_ref[...]
    dinv = lax.rsqrt(cnt + 1.0)
    h = jnp.dot(x_ref[...], w1_ref[...], preferred_element_type=jnp.float32)
    g_ref[...] = h * dinv
    dinv_ref[...] = dinv


def _tc_b(xp, w1, c0, c1):
    return pl.pallas_call(
        _tc_b_body,
        grid=(GRID,),
        in_specs=[
            pl.BlockSpec((BM, D), lambda i: (i, 0)),
            pl.BlockSpec((D, D), lambda i: (0, 0)),
            pl.BlockSpec((BM, 1), lambda i: (i, 0)),
            pl.BlockSpec((BM, 1), lambda i: (i, 0)),
        ],
        out_specs=[
            pl.BlockSpec((BM, D), lambda i: (i, 0)),
            pl.BlockSpec((BM, 1), lambda i: (i, 0)),
        ],
        out_shape=[
            jax.ShapeDtypeStruct((NP, D), jnp.float32),
            jax.ShapeDtypeStruct((NP, 1), jnp.float32),
        ],
    )(xp, w1, c0, c1)


def _tc_d_body(a0_ref, a1_ref, g_ref, dinv_ref, b1_ref, w2_ref, t_ref):
    acc = a0_ref[...] + a1_ref[...] + g_ref[...]
    out1 = acc * dinv_ref[...] + b1_ref[...]
    h1 = jnp.maximum(out1, 0.0)
    s = jnp.dot(h1, w2_ref[...], preferred_element_type=jnp.float32)
    t_ref[...] = s * dinv_ref[...]


def _tc_d(a0, a1, g, dinv, b1r, w2):
    return pl.pallas_call(
        _tc_d_body,
        grid=(GRID,),
        in_specs=[
            pl.BlockSpec((BM, D), lambda i: (i, 0)),
            pl.BlockSpec((BM, D), lambda i: (i, 0)),
            pl.BlockSpec((BM, D), lambda i: (i, 0)),
            pl.BlockSpec((BM, 1), lambda i: (i, 0)),
            pl.BlockSpec((1, D), lambda i: (0, 0)),
            pl.BlockSpec((D, 1), lambda i: (0, 0)),
        ],
        out_specs=pl.BlockSpec((BM, 1), lambda i: (i, 0)),
        out_shape=jax.ShapeDtypeStruct((NP, 1), jnp.float32),
    )(a0, a1, g, dinv, b1r, w2)


def _tc_f_body(q0_ref, q1_ref, t_ref, dinv_ref, b2_ref, o_ref):
    o_ref[...] = (q0_ref[...] + q1_ref[...] + t_ref[...]) * dinv_ref[...] \
        + b2_ref[...]


def _tc_f(q0, q1, t, dinv, b2r):
    return pl.pallas_call(
        _tc_f_body,
        grid=(GRID,),
        in_specs=[
            pl.BlockSpec((BM, 1), lambda i: (i, 0)),
            pl.BlockSpec((BM, 1), lambda i: (i, 0)),
            pl.BlockSpec((BM, 1), lambda i: (i, 0)),
            pl.BlockSpec((BM, 1), lambda i: (i, 0)),
            pl.BlockSpec((1, 1), lambda i: (0, 0)),
        ],
        out_specs=pl.BlockSpec((BM, 1), lambda i: (i, 0)),
        out_shape=jax.ShapeDtypeStruct((NP, 1), jnp.float32),
    )(q0, q1, t, dinv, b2r)


def kernel(x, edge_index, edge_attr, W1, b1, W2, b2):
    ei = edge_index.astype(jnp.int32)
    pad = N + jnp.arange(EP - E, dtype=jnp.int32) % (NP - N)
    srcp = jnp.concatenate([ei[0], pad])
    dstp = jnp.concatenate([ei[1], pad])
    src_a = srcp.reshape(NW, NCHA, CH)
    dst_a = dstp.reshape(NW, NCHA, CH)
    src_c = srcp.reshape(NW, NST, SI, CHC)
    dst_c = dstp.reshape(NW, NST, SI, CHC)
    xp = jnp.zeros((NP, D), jnp.float32).at[:N].set(x)

    cnt = _sc_count(dst_a)
    c0 = cnt[0].reshape(NP, 1)
    c1 = cnt[1].reshape(NP, 1)
    g, dinv = _tc_b(xp, W1, c0, c1)
    acc = _sc_agg(src_c, dst_c, g)
    t = _tc_d(acc[0], acc[1], g, dinv, b1.reshape(1, D), W2)
    q = _sc_agg1(src_a, dst_a, t.reshape(NP))
    out = _tc_f(q[0].reshape(NP, 1), q[1].reshape(NP, 1), t, dinv,
                b2.reshape(1, 1))
    return out[:N]


# speedup vs baseline: 1.1856x; 1.1400x over previous
import functools

import jax
import jax.numpy as jnp
from jax import lax
from jax.experimental import pallas as pl
from jax.experimental.pallas import tpu as pltpu
from jax.experimental.pallas import tpu_sc as plsc

N = 10000
NP = 10240
D = 128
E = 320000
EP = 327680
NC = 2
NS = 16
NW = NC * NS
CH = 128
EPT = EP // NW
NCHA = EPT // CH
CHC = 64
NT = EPT // CHC
SI = 10
NST = NT // SI
NB = 4
NBE = 8
RPT = NP // NS

_mesh = plsc.VectorSubcoreMesh(
    core_axis_name="c", subcore_axis_name="s", num_cores=NC, num_subcores=NS)


@functools.partial(
    pl.kernel,
    out_type=jax.ShapeDtypeStruct((NC, NP), jnp.float32),
    mesh=_mesh,
    scratch_types=[
        pltpu.VMEM((NCHA, CH), jnp.int32),
        pltpu.VMEM((CH,), jnp.float32),
        pltpu.VMEM((RPT,), jnp.float32),
        pltpu.VMEM_SHARED((NP,), jnp.float32),
        pltpu.SemaphoreType.DMA,
    ],
)
def _sc_count(dst_h, cnt_h, idx_v, ones_v, zv, cnt_sh, sem_s):
    cid = lax.axis_index("c")
    sid = lax.axis_index("s")
    wid = cid * NS + sid
    for i in range(CH // 16):
        ones_v[pl.ds(i * 16, 16)] = jnp.ones((16,), jnp.float32)
    for i in range(RPT // 16):
        zv[pl.ds(i * 16, 16)] = jnp.zeros((16,), jnp.float32)
    pltpu.sync_copy(zv, cnt_sh.at[pl.ds(sid * RPT, RPT)])
    plsc.subcore_barrier()
    pltpu.sync_copy(dst_h.at[wid], idx_v)

    def chunk(c, carry):
        @pl.when(c >= 8)
        def _drain():
            pltpu.make_async_copy(ones_v, cnt_sh.at[idx_v.at[0]],
                                  sem_s).wait()

        pltpu.async_copy(ones_v, cnt_sh.at[idx_v.at[c]], sem_s, add=True)
        return carry

    lax.fori_loop(0, NCHA, chunk, 0)
    for _ in range(8):
        pltpu.make_async_copy(ones_v, cnt_sh.at[idx_v.at[0]], sem_s).wait()
    plsc.subcore_barrier()
    pltpu.sync_copy(cnt_sh.at[pl.ds(sid * RPT, RPT)],
                    cnt_h.at[cid, pl.ds(sid * RPT, RPT)])


@functools.partial(
    pl.kernel,
    out_type=jax.ShapeDtypeStruct((NC, NP, D), jnp.float32),
    mesh=_mesh,
    scratch_types=[
        pltpu.VMEM((2, SI, CHC), jnp.int32),
        pltpu.VMEM((2, SI, CHC), jnp.int32),
        pltpu.VMEM((NB, CHC, D), jnp.float32),
        pltpu.VMEM_SHARED((NP, D), jnp.float32),
        pltpu.SemaphoreType.DMA,
        pltpu.SemaphoreType.DMA,
        pltpu.SemaphoreType.DMA,
    ],
)
def _sc_agg(src_h, dst_h, g_h, acc_h, isrc, idst, rows, acc_sh,
            sem_g, sem_s, sem_i):
    cid = lax.axis_index("c")
    sid = lax.axis_index("s")
    wid = cid * NS + sid

    def zbody(i, carry):
        for j in range(D // 16):
            rows[0, i, pl.ds(j * 16, 16)] = jnp.zeros((16,), jnp.float32)
        return carry

    lax.fori_loop(0, CHC, zbody, 0)
    for k in range(RPT // CHC):
        pltpu.sync_copy(rows.at[0], acc_sh.at[pl.ds(sid * RPT + k * CHC, CHC)])
    plsc.subcore_barrier()

    pltpu.sync_copy(src_h.at[wid, 0], isrc.at[0])
    pltpu.sync_copy(dst_h.at[wid, 0], idst.at[0])
    pltpu.async_copy(g_h.at[isrc.at[0, 0]], rows.at[0], sem_g)
    pltpu.async_copy(g_h.at[isrc.at[0, 1]], rows.at[1], sem_g)

    def step(c, carry):
        st = c // SI
        cc = c % SI
        sb = st % 2

        @pl.when(jnp.logical_and(cc == 0, st + 1 < NST))
        def _pref():
            pltpu.async_copy(src_h.at[wid, st + 1],
                             isrc.at[(st + 1) % 2], sem_i)
            pltpu.async_copy(dst_h.at[wid, st + 1],
                             idst.at[(st + 1) % 2], sem_i)

        @pl.when(c >= 2)
        def _drain():
            pltpu.make_async_copy(rows.at[0], acc_sh.at[idst.at[0, 0]],
                                  sem_s).wait()

        pltpu.make_async_copy(g_h.at[pl.ds(0, CHC)], rows.at[c % NB],
                              sem_g).wait()

        @pl.when(jnp.logical_and(cc == SI - 2, st + 1 < NST))
        def _wslab():
            pltpu.make_async_copy(src_h.at[0, 0], isrc.at[0], sem_i).wait()
            pltpu.make_async_copy(dst_h.at[0, 0], idst.at[0], sem_i).wait()

        @pl.when(c + 2 < NT)
        def _gnext():
            c2 = c + 2
            pltpu.async_copy(
                g_h.at[isrc.at[(c2 // SI) % 2, c2 % SI]],
                rows.at[c2 % NB], sem_g)

        pltpu.async_copy(rows.at[c % NB], acc_sh.at[idst.at[sb, cc]],
                         sem_s, add=True)
        return carry

    lax.fori_loop(0, NT, step, 0, unroll=2)
    for _ in range(2):
        pltpu.make_async_copy(rows.at[0], acc_sh.at[idst.at[0, 0]],
                              sem_s).wait()
    plsc.subcore_barrier()
    pltpu.sync_copy(acc_sh.at[pl.ds(sid * RPT, RPT)],
                    acc_h.at[cid, pl.ds(sid * RPT, RPT)])


@functools.partial(
    pl.kernel,
    out_type=jax.ShapeDtypeStruct((NC, NP), jnp.float32),
    mesh=_mesh,
    scratch_types=[
        pltpu.VMEM((NCHA, CH), jnp.int32),
        pltpu.VMEM((NCHA, CH), jnp.int32),
        pltpu.VMEM((NBE, CH), jnp.float32),
        pltpu.VMEM((RPT,), jnp.float32),
        pltpu.VMEM_SHARED((NP,), jnp.float32),
        pltpu.VMEM_SHARED((NP,), jnp.float32),
        pltpu.SemaphoreType.DMA,
        pltpu.SemaphoreType.DMA,
    ],
)
def _sc_agg1(src_h, dst_h, t_h, acc_h, isrc, idst, tv, zv, acc_sh, t_sp,
             sem_g, sem_s):
    cid = lax.axis_index("c")
    sid = lax.axis_index("s")
    wid = cid * NS + sid
    for i in range(RPT // 16):
        zv[pl.ds(i * 16, 16)] = jnp.zeros((16,), jnp.float32)
    pltpu.sync_copy(zv, acc_sh.at[pl.ds(sid * RPT, RPT)])
    pltpu.sync_copy(t_h.at[pl.ds(sid * RPT, RPT)],
                    t_sp.at[pl.ds(sid * RPT, RPT)])
    plsc.subcore_barrier()
    pltpu.sync_copy(src_h.at[wid], isrc)
    pltpu.sync_copy(dst_h.at[wid], idst)
    pltpu.async_copy(t_sp.at[isrc.at[0]], tv.at[0], sem_g)
    pltpu.async_copy(t_sp.at[isrc.at[1]], tv.at[1], sem_g)

    def step(c, carry):
        @pl.when(c >= NBE - 2)
        def _drain():
            pltpu.make_async_copy(tv.at[0], acc_sh.at[idst.at[0]],
                                  sem_s).wait()

        pltpu.make_async_copy(t_h.at[pl.ds(0, CH)], tv.at[c % NBE],
                              sem_g).wait()

        @pl.when(c + 2 < NCHA)
        def _gnext():
            pltpu.async_copy(t_sp.at[isrc.at[c + 2]], tv.at[(c + 2) % NBE],
                             sem_g)

        pltpu.async_copy(tv.at[c % NBE], acc_sh.at[idst.at[c]],
                         sem_s, add=True)
        return carry

    lax.fori_loop(0, NCHA, step, 0, unroll=2)
    for _ in range(NBE - 2):
        pltpu.make_async_copy(tv.at[0], acc_sh.at[idst.at[0]], sem_s).wait()
    plsc.subcore_barrier()
    pltpu.sync_copy(acc_sh.at[pl.ds(sid * RPT, RPT)],
                    acc_h.at[cid, pl.ds(sid * RPT, RPT)])


BM = 640
GRID = NP // BM


def _tc_b_body(x_ref, w1_ref, c0_ref, c1_ref, g_ref, dinv_ref):
    cnt = c0_ref[...] + c1_ref[...]
    dinv = lax.rsqrt(cnt + 1.0)
    h = jnp.dot(x_ref[...], w1_ref[...], preferred_element_type=jnp.float32)
    g_ref[...] = h * dinv
    dinv_ref[...] = dinv


def _tc_b(xp, w1, c0, c1):
    return pl.pallas_call(
        _tc_b_body,
        grid=(GRID,),
        in_specs=[
            pl.BlockSpec((BM, D), lambda i: (i, 0)),
            pl.BlockSpec((D, D), lambda i: (0, 0)),
            pl.BlockSpec((BM, 1), lambda i: (i, 0)),
            pl.BlockSpec((BM, 1), lambda i: (i, 0)),
        ],
        out_specs=[
            pl.BlockSpec((BM, D), lambda i: (i, 0)),
            pl.BlockSpec((BM, 1), lambda i: (i, 0)),
        ],
        out_shape=[
            jax.ShapeDtypeStruct((NP, D), jnp.float32),
            jax.ShapeDtypeStruct((NP, 1), jnp.float32),
        ],
    )(xp, w1, c0, c1)


def _tc_d_body(a0_ref, a1_ref, g_ref, dinv_ref, b1_ref, w2_ref, t_ref):
    acc = a0_ref[...] + a1_ref[...] + g_ref[...]
    out1 = acc * dinv_ref[...] + b1_ref[...]
    h1 = jnp.maximum(out1, 0.0)
    s = jnp.dot(h1, w2_ref[...], preferred_element_type=jnp.float32)
    t_ref[...] = s * dinv_ref[...]


def _tc_d(a0, a1, g, dinv, b1r, w2):
    return pl.pallas_call(
        _tc_d_body,
        grid=(GRID,),
        in_specs=[
            pl.BlockSpec((BM, D), lambda i: (i, 0)),
            pl.BlockSpec((BM, D), lambda i: (i, 0)),
            pl.BlockSpec((BM, D), lambda i: (i, 0)),
            pl.BlockSpec((BM, 1), lambda i: (i, 0)),
            pl.BlockSpec((1, D), lambda i: (0, 0)),
            pl.BlockSpec((D, 1), lambda i: (0, 0)),
        ],
        out_specs=pl.BlockSpec((BM, 1), lambda i: (i, 0)),
        out_shape=jax.ShapeDtypeStruct((NP, 1), jnp.float32),
    )(a0, a1, g, dinv, b1r, w2)


def _tc_f_body(q0_ref, q1_ref, t_ref, dinv_ref, b2_ref, o_ref):
    o_ref[...] = (q0_ref[...] + q1_ref[...] + t_ref[...]) * dinv_ref[...] \
        + b2_ref[...]


def _tc_f(q0, q1, t, dinv, b2r):
    return pl.pallas_call(
        _tc_f_body,
        grid=(GRID,),
        in_specs=[
            pl.BlockSpec((BM, 1), lambda i: (i, 0)),
            pl.BlockSpec((BM, 1), lambda i: (i, 0)),
            pl.BlockSpec((BM, 1), lambda i: (i, 0)),
            pl.BlockSpec((BM, 1), lambda i: (i, 0)),
            pl.BlockSpec((1, 1), lambda i: (0, 0)),
        ],
        out_specs=pl.BlockSpec((BM, 1), lambda i: (i, 0)),
        out_shape=jax.ShapeDtypeStruct((NP, 1), jnp.float32),
    )(q0, q1, t, dinv, b2r)


def kernel(x, edge_index, edge_attr, W1, b1, W2, b2):
    ei = edge_index.astype(jnp.int32)
    pad = N + jnp.arange(EP - E, dtype=jnp.int32) % (NP - N)
    srcp = jnp.concatenate([ei[0], pad])
    dstp = jnp.concatenate([ei[1], pad])
    src_a = srcp.reshape(NW, NCHA, CH)
    src_e = srcp.reshape(NW, EPT)
    dst_a = dstp.reshape(NW, NCHA, CH)
    src_c = srcp.reshape(NW, NST, SI, CHC)
    dst_c = dstp.reshape(NW, NST, SI, CHC)
    xp = jnp.zeros((NP, D), jnp.float32).at[:N].set(x)

    cnt = _sc_count(dst_a)
    c0 = cnt[0].reshape(NP, 1)
    c1 = cnt[1].reshape(NP, 1)
    g, dinv = _tc_b(xp, W1, c0, c1)
    acc = _sc_agg(src_c, dst_c, g)
    t = _tc_d(acc[0], acc[1], g, dinv, b1.reshape(1, D), W2)
    q = _sc_agg1(src_a, dst_a, t.reshape(NP))
    out = _tc_f(q[0].reshape(NP, 1), q[1].reshape(NP, 1), t, dinv,
                b2.reshape(1, 1))
    return out[:N]


# speedup vs baseline: 1.2902x; 1.0883x over previous
import functools

import jax
import jax.numpy as jnp
from jax import lax
from jax.experimental import pallas as pl
from jax.experimental.pallas import tpu as pltpu
from jax.experimental.pallas import tpu_sc as plsc

N = 10000
NP = 10240
D = 128
E = 320000
EP = 327680
NC = 2
NS = 16
NW = NC * NS
CH = 128
EPT = EP // NW
NCHA = EPT // CH
CHC = 80
NT = EPT // CHC
SI = 8
NST = NT // SI
NB = 4
NBE = 8
ETE = EP // NS
NTE = ETE // CH
RPT = NP // NS
RPE = NP // NS // NC

_mesh = plsc.VectorSubcoreMesh(
    core_axis_name="c", subcore_axis_name="s", num_cores=NC, num_subcores=NS)


@functools.partial(
    pl.kernel,
    out_type=jax.ShapeDtypeStruct((NC, NP), jnp.float32),
    mesh=_mesh,
    scratch_types=[
        pltpu.VMEM((NCHA, CH), jnp.int32),
        pltpu.VMEM((CH,), jnp.float32),
        pltpu.VMEM((RPT,), jnp.float32),
        pltpu.VMEM_SHARED((NP,), jnp.float32),
        pltpu.SemaphoreType.DMA,
    ],
)
def _sc_count(dst_h, cnt_h, idx_v, ones_v, zv, cnt_sh, sem_s):
    cid = lax.axis_index("c")
    sid = lax.axis_index("s")
    wid = cid * NS + sid
    for i in range(CH // 16):
        ones_v[pl.ds(i * 16, 16)] = jnp.ones((16,), jnp.float32)
    for i in range(RPT // 16):
        zv[pl.ds(i * 16, 16)] = jnp.zeros((16,), jnp.float32)
    pltpu.sync_copy(zv, cnt_sh.at[pl.ds(sid * RPT, RPT)])
    plsc.subcore_barrier()
    pltpu.sync_copy(dst_h.at[wid], idx_v)

    def chunk(c, carry):
        @pl.when(c >= 8)
        def _drain():
            pltpu.make_async_copy(ones_v, cnt_sh.at[idx_v.at[0]],
                                  sem_s).wait()

        pltpu.async_copy(ones_v, cnt_sh.at[idx_v.at[c]], sem_s, add=True)
        return carry

    lax.fori_loop(0, NCHA, chunk, 0)
    for _ in range(8):
        pltpu.make_async_copy(ones_v, cnt_sh.at[idx_v.at[0]], sem_s).wait()
    plsc.subcore_barrier()
    pltpu.sync_copy(cnt_sh.at[pl.ds(sid * RPT, RPT)],
                    cnt_h.at[cid, pl.ds(sid * RPT, RPT)])


@functools.partial(
    pl.kernel,
    out_type=jax.ShapeDtypeStruct((NC, NP, D), jnp.float32),
    mesh=_mesh,
    scratch_types=[
        pltpu.VMEM((2, SI, CHC), jnp.int32),
        pltpu.VMEM((2, SI, CHC), jnp.int32),
        pltpu.VMEM((NB, CHC, D), jnp.float32),
        pltpu.VMEM_SHARED((NP, D), jnp.float32),
        pltpu.SemaphoreType.DMA,
        pltpu.SemaphoreType.DMA,
        pltpu.SemaphoreType.DMA,
    ],
)
def _sc_agg(src_h, dst_h, g_h, acc_h, isrc, idst, rows, acc_sh,
            sem_g, sem_s, sem_i):
    cid = lax.axis_index("c")
    sid = lax.axis_index("s")
    wid = cid * NS + sid

    def zbody(i, carry):
        for j in range(D // 16):
            rows[0, i, pl.ds(j * 16, 16)] = jnp.zeros((16,), jnp.float32)
        return carry

    lax.fori_loop(0, CHC, zbody, 0)
    for k in range(RPT // CHC):
        pltpu.sync_copy(rows.at[0], acc_sh.at[pl.ds(sid * RPT + k * CHC, CHC)])
    plsc.subcore_barrier()

    pltpu.sync_copy(src_h.at[wid, 0], isrc.at[0])
    pltpu.sync_copy(dst_h.at[wid, 0], idst.at[0])
    pltpu.async_copy(g_h.at[isrc.at[0, 0]], rows.at[0], sem_g)
    pltpu.async_copy(g_h.at[isrc.at[0, 1]], rows.at[1], sem_g)

    def step(c, carry):
        st = c // SI
        cc = c % SI
        sb = st % 2

        @pl.when(jnp.logical_and(cc == 0, st + 1 < NST))
        def _pref():
            pltpu.async_copy(src_h.at[wid, st + 1],
                             isrc.at[(st + 1) % 2], sem_i)
            pltpu.async_copy(dst_h.at[wid, st + 1],
                             idst.at[(st + 1) % 2], sem_i)

        @pl.when(c >= 2)
        def _drain():
            pltpu.make_async_copy(rows.at[0], acc_sh.at[idst.at[0, 0]],
                                  sem_s).wait()

        pltpu.make_async_copy(g_h.at[pl.ds(0, CHC)], rows.at[c % NB],
                              sem_g).wait()

        @pl.when(jnp.logical_and(cc == SI - 2, st + 1 < NST))
        def _wslab():
            pltpu.make_async_copy(src_h.at[0, 0], isrc.at[0], sem_i).wait()
            pltpu.make_async_copy(dst_h.at[0, 0], idst.at[0], sem_i).wait()

        @pl.when(c + 2 < NT)
        def _gnext():
            c2 = c + 2
            pltpu.async_copy(
                g_h.at[isrc.at[(c2 // SI) % 2, c2 % SI]],
                rows.at[c2 % NB], sem_g)

        pltpu.async_copy(rows.at[c % NB], acc_sh.at[idst.at[sb, cc]],
                         sem_s, add=True)
        return carry

    lax.fori_loop(0, NT, step, 0, unroll=2)
    for _ in range(2):
        pltpu.make_async_copy(rows.at[0], acc_sh.at[idst.at[0, 0]],
                              sem_s).wait()
    plsc.subcore_barrier()
    pltpu.sync_copy(acc_sh.at[pl.ds(sid * RPT, RPT)],
                    acc_h.at[cid, pl.ds(sid * RPT, RPT)])


@functools.partial(
    pl.kernel,
    out_type=jax.ShapeDtypeStruct((NP,), jnp.float32),
    mesh=_mesh,
    scratch_types=[
        pltpu.VMEM((NTE, CH), jnp.int32),
        pltpu.VMEM((NTE, CH), jnp.int32),
        pltpu.VMEM((NBE, CH), jnp.float32),
        pltpu.VMEM((RPT,), jnp.float32),
        pltpu.VMEM((RPE,), jnp.float32),
        pltpu.VMEM((RPE,), jnp.float32),
        pltpu.VMEM((RPE,), jnp.float32),
        pltpu.VMEM((16,), jnp.float32),
        pltpu.VMEM_SHARED((NP,), jnp.float32),
        pltpu.VMEM_SHARED((NP,), jnp.float32),
        pltpu.SemaphoreType.DMA,
        pltpu.SemaphoreType.DMA,
    ],
)
def _sc_agg1(src_h, dst_h, t_h, dinv_h, b2_h, out_h, isrc, idst, tv, zv,
             qv, dv, ov, bv, acc_sh, t_sp, sem_g, sem_s):
    cid = lax.axis_index("c")
    sid = lax.axis_index("s")
    for i in range(RPT // 16):
        zv[pl.ds(i * 16, 16)] = jnp.zeros((16,), jnp.float32)
    pltpu.sync_copy(zv, acc_sh.at[pl.ds(sid * RPT, RPT)])
    pltpu.sync_copy(t_h.at[pl.ds(sid * RPT, RPT)],
                    t_sp.at[pl.ds(sid * RPT, RPT)])
    plsc.subcore_barrier()
    pltpu.sync_copy(src_h.at[sid], isrc)
    pltpu.sync_copy(dst_h.at[sid], idst)
    pltpu.async_copy(t_sp.at[isrc.at[0]], tv.at[0], sem_g)
    pltpu.async_copy(t_sp.at[isrc.at[1]], tv.at[1], sem_g)

    def step(c, carry):
        @pl.when(c >= NBE - 2)
        def _drain():
            pltpu.make_async_copy(tv.at[0], acc_sh.at[idst.at[0]],
                                  sem_s).wait()

        pltpu.make_async_copy(t_h.at[pl.ds(0, CH)], tv.at[c % NBE],
                              sem_g).wait()

        @pl.when(c + 2 < NTE)
        def _gnext():
            pltpu.async_copy(t_sp.at[isrc.at[c + 2]], tv.at[(c + 2) % NBE],
                             sem_g)

        pltpu.async_copy(tv.at[c % NBE], acc_sh.at[idst.at[c]],
                         sem_s, add=True)
        return carry

    lax.fori_loop(0, NTE, step, 0, unroll=2)
    for _ in range(NBE - 2):
        pltpu.make_async_copy(tv.at[0], acc_sh.at[idst.at[0]], sem_s).wait()
    plsc.subcore_barrier()

    base = cid * (NP // NC) + sid * RPE
    pltpu.sync_copy(acc_sh.at[pl.ds(base, RPE)], qv)
    pltpu.sync_copy(t_sp.at[pl.ds(base, RPE)], zv.at[pl.ds(0, RPE)])
    pltpu.sync_copy(dinv_h.at[pl.ds(base, RPE)], dv)
    pltpu.sync_copy(b2_h, bv)
    b2 = bv[pl.ds(0, 16)]
    for i in range(RPE // 16):
        sl = pl.ds(i * 16, 16)
        ov[sl] = dv[sl] * (qv[sl] + zv[sl]) + b2
    pltpu.sync_copy(ov, out_h.at[pl.ds(base, RPE)])


BM = 640
GRID = NP // BM


def _tc_b_body(x_ref, w1_ref, c0_ref, c1_ref, g_ref, dinv_ref):
    cnt = c0_ref[...] + c1_ref[...]
    dinv = lax.rsqrt(cnt + 1.0)
    h = jnp.dot(x_ref[...], w1_ref[...], preferred_element_type=jnp.float32)
    g_ref[...] = h * dinv
    dinv_ref[...] = dinv


def _tc_b(xp, w1, c0, c1):
    return pl.pallas_call(
        _tc_b_body,
        grid=(GRID,),
        in_specs=[
            pl.BlockSpec((BM, D), lambda i: (i, 0)),
            pl.BlockSpec((D, D), lambda i: (0, 0)),
            pl.BlockSpec((BM, 1), lambda i: (i, 0)),
            pl.BlockSpec((BM, 1), lambda i: (i, 0)),
        ],
        out_specs=[
            pl.BlockSpec((BM, D), lambda i: (i, 0)),
            pl.BlockSpec((BM, 1), lambda i: (i, 0)),
        ],
        out_shape=[
            jax.ShapeDtypeStruct((NP, D), jnp.float32),
            jax.ShapeDtypeStruct((NP, 1), jnp.float32),
        ],
    )(xp, w1, c0, c1)


def _tc_d_body(a0_ref, a1_ref, g_ref, dinv_ref, b1_ref, w2_ref, t_ref):
    acc = a0_ref[...] + a1_ref[...] + g_ref[...]
    out1 = acc * dinv_ref[...] + b1_ref[...]
    h1 = jnp.maximum(out1, 0.0)
    s = jnp.dot(h1, w2_ref[...], preferred_element_type=jnp.float32)
    t_ref[...] = s * dinv_ref[...]


def _tc_d(a0, a1, g, dinv, b1r, w2):
    return pl.pallas_call(
        _tc_d_body,
        grid=(GRID,),
        in_specs=[
            pl.BlockSpec((BM, D), lambda i: (i, 0)),
            pl.BlockSpec((BM, D), lambda i: (i, 0)),
            pl.BlockSpec((BM, D), lambda i: (i, 0)),
            pl.BlockSpec((BM, 1), lambda i: (i, 0)),
            pl.BlockSpec((1, D), lambda i: (0, 0)),
            pl.BlockSpec((D, 1), lambda i: (0, 0)),
        ],
        out_specs=pl.BlockSpec((BM, 1), lambda i: (i, 0)),
        out_shape=jax.ShapeDtypeStruct((NP, 1), jnp.float32),
    )(a0, a1, g, dinv, b1r, w2)


def kernel(x, edge_index, edge_attr, W1, b1, W2, b2):
    ei = edge_index.astype(jnp.int32)
    pad = N + jnp.arange(EP - E, dtype=jnp.int32) % (NP - N)
    srcp = jnp.concatenate([ei[0], pad])
    dstp = jnp.concatenate([ei[1], pad])
    src_a = srcp.reshape(NW, NCHA, CH)
    dst_a = dstp.reshape(NW, NCHA, CH)
    src_c = srcp.reshape(NW, NST, SI, CHC)
    dst_c = dstp.reshape(NW, NST, SI, CHC)
    src_e = srcp.reshape(NS, NTE, CH)
    dst_e = dstp.reshape(NS, NTE, CH)
    xp = jnp.zeros((NP, D), jnp.float32).at[:N].set(x)

    cnt = _sc_count(dst_a)
    c0 = cnt[0].reshape(NP, 1)
    c1 = cnt[1].reshape(NP, 1)
    g, dinv = _tc_b(xp, W1, c0, c1)
    acc = _sc_agg(src_c, dst_c, g)
    t = _tc_d(acc[0], acc[1], g, dinv, b1.reshape(1, D), W2)
    out = _sc_agg1(src_e, dst_e, t.reshape(NP), dinv.reshape(NP),
                   jnp.broadcast_to(b2, (16,)))
    return out[:N].reshape(N, 1)
